# Initial kernel scaffold; baseline (speedup 1.0000x reference)
import jax
import jax.numpy as jnp
from jax.experimental import pallas as pl


def kernel(y_true, embeddings, src, dst):
    raise NotImplementedError("write your pallas kernel here")


# speedup vs baseline: 1.8994x; 1.8994x over previous
import functools

import jax
import jax.numpy as jnp
from jax import lax
from jax.experimental import pallas as pl
from jax.experimental.pallas import tpu as pltpu
from jax.experimental.pallas import tpu_sc as plsc

N = 100000
D = 128
NC = 2
NS = 16
NW = NC * NS
CHUNK = 80
NB = CHUNK // 16
NCHUNKS = N // CHUNK
NITER = (NCHUNKS + NW - 1) // NW
NPAD = 100096
NROWS = NPAD // 128


def _sc_body(y_hbm, dst_hbm, emb_hbm, dout, mout, pout,
             idx_v, ei_v, ej_v, yi_v, yj_v, dbuf, mbuf,
             acc_s, acc_z, stage, zbuf, in_sem, wb_sem):
    wid = lax.axis_index("s") * NC + lax.axis_index("c")
    zero16 = jnp.zeros((16,), jnp.float32)
    acc_s[...] = zero16
    acc_z[...] = zero16
    lane = jnp.arange(16, dtype=jnp.int32)

    def issue(t, p):
        c = wid + t * NW

        @pl.when(c < NCHUNKS)
        def _():
            base = c * CHUNK
            pltpu.sync_copy(dst_hbm.at[pl.ds(base, CHUNK)], idx_v[p])
            pltpu.async_copy(emb_hbm.at[idx_v[p]], ej_v[p], in_sem[p])
            pltpu.async_copy(y_hbm.at[idx_v[p]], yj_v[p], in_sem[p])
            pltpu.async_copy(emb_hbm.at[pl.ds(base, CHUNK)], ei_v[p],
                             in_sem[p])
            pltpu.async_copy(y_hbm.at[pl.ds(base, CHUNK)], yi_v[p], in_sem[p])

    def wait_inputs(p):
        pltpu.make_async_copy(emb_hbm.at[pl.ds(0, CHUNK)], ej_v[p],
                              in_sem[p]).wait()
        pltpu.make_async_copy(y_hbm.at[pl.ds(0, CHUNK)], yj_v[p],
                              in_sem[p]).wait()
        pltpu.make_async_copy(emb_hbm.at[pl.ds(0, CHUNK)], ei_v[p],
                              in_sem[p]).wait()
        pltpu.make_async_copy(y_hbm.at[pl.ds(0, CHUNK)], yi_v[p],
                              in_sem[p]).wait()

    def compute(t, p):
        c = wid + t * NW

        @pl.when(c < NCHUNKS)
        def _():
            base = c * CHUNK
            wait_inputs(p)

            @pl.when(t >= 2)
            def _():
                pltpu.make_async_copy(dout.at[pl.ds(0, CHUNK)], dbuf[p],
                                      wb_sem[p]).wait()
                pltpu.make_async_copy(mout.at[pl.ds(0, CHUNK)], mbuf[p],
                                      wb_sem[p]).wait()

            for b in range(NB):
                rowi = lane + (b * 16)

                def kstep(k, car):
                    accd, accz = car
                    col = (lane + k) & (D - 1)
                    vi = plsc.load_gather(ei_v[p], [rowi, col])
                    vj = plsc.load_gather(ej_v[p], [rowi, col])
                    dl = vi - vj
                    return accd + dl * dl, accz + vi * vi

                accd, accz = lax.fori_loop(0, D, kstep, (zero16, zero16),
                                           unroll=8)
                yi_b = yi_v[p][pl.ds(b * 16, 16)]
                yj_b = yj_v[p][pl.ds(b * 16, 16)]
                same = yi_b == yj_b
                acc_s[...] = acc_s[...] + jnp.where(same, accd, 0.0)
                acc_z[...] = acc_z[...] + jnp.where(yi_b == 0, accz, 0.0)
                m = jnp.where(same, 0.0,
                              jnp.abs(yi_b - yj_b).astype(jnp.float32))
                dbuf[p][pl.ds(b * 16, 16)] = accd
                mbuf[p][pl.ds(b * 16, 16)] = m

            pltpu.async_copy(dbuf[p], dout.at[pl.ds(base, CHUNK)], wb_sem[p])
            pltpu.async_copy(mbuf[p], mout.at[pl.ds(base, CHUNK)], wb_sem[p])

    issue(0, 0)

    def pair_body(t2, carry):
        for p in range(2):
            t = t2 + p
            issue(t + 1, 1 - p)
            compute(t, p)
        return carry

    lax.fori_loop(0, NITER // 2, lambda i, cr: pair_body(i * 2, cr), 0)

    for p in range(2):
        pltpu.make_async_copy(dout.at[pl.ds(0, CHUNK)], dbuf[p],
                              wb_sem[p]).wait()
        pltpu.make_async_copy(mout.at[pl.ds(0, CHUNK)], mbuf[p],
                              wb_sem[p]).wait()

    stage[0, :] = acc_s[...]
    stage[1, :] = acc_z[...]
    pltpu.sync_copy(stage, pout.at[wid])

    @pl.when(wid == 0)
    def _():
        for j in range(6):
            zbuf[pl.ds(j * 16, 16)] = jnp.zeros((16,), jnp.float32)
        pltpu.sync_copy(zbuf, dout.at[pl.ds(N, NPAD - N)])
        pltpu.sync_copy(zbuf, mout.at[pl.ds(N, NPAD - N)])


_sc_kernel = functools.partial(
    pl.kernel,
    compiler_params=pltpu.CompilerParams(needs_layout_passes=False),
    out_type=(
        jax.ShapeDtypeStruct((NPAD,), jnp.float32),
        jax.ShapeDtypeStruct((NPAD,), jnp.float32),
        jax.ShapeDtypeStruct((NW, 2, 16), jnp.float32),
    ),
    mesh=plsc.VectorSubcoreMesh(core_axis_name="c", subcore_axis_name="s",
                                num_cores=NC, num_subcores=NS),
    scratch_types=(
        [pltpu.VMEM((CHUNK,), jnp.int32)] * 2,
        [pltpu.VMEM((CHUNK, D), jnp.float32)] * 2,
        [pltpu.VMEM((CHUNK, D), jnp.float32)] * 2,
        [pltpu.VMEM((CHUNK,), jnp.int32)] * 2,
        [pltpu.VMEM((CHUNK,), jnp.int32)] * 2,
        [pltpu.VMEM((CHUNK,), jnp.float32)] * 2,
        [pltpu.VMEM((CHUNK,), jnp.float32)] * 2,
        pltpu.VMEM((16,), jnp.float32),
        pltpu.VMEM((16,), jnp.float32),
        pltpu.VMEM((2, 16), jnp.float32),
        pltpu.VMEM((NPAD - N,), jnp.float32),
        [pltpu.SemaphoreType.DMA] * 2,
        [pltpu.SemaphoreType.DMA] * 2,
    ),
)(_sc_body)


def _tc_body(d_ref, m_ref, p_ref, out_ref):
    d = d_ref[...]
    m = m_ref[...]
    t = jnp.maximum(m - jnp.sqrt(d + 1e-6), 0.0)
    out_ref[0, 0] = jnp.sum(t * t) + jnp.sum(p_ref[...])


_tc_kernel = pl.pallas_call(
    _tc_body,
    out_shape=jax.ShapeDtypeStruct((1, 1), jnp.float32),
    in_specs=[
        pl.BlockSpec(memory_space=pltpu.VMEM),
        pl.BlockSpec(memory_space=pltpu.VMEM),
        pl.BlockSpec(memory_space=pltpu.VMEM),
    ],
    out_specs=pl.BlockSpec(memory_space=pltpu.SMEM),
)


def kernel(y_true, embeddings, src, dst):
    y = y_true.astype(jnp.int32)
    dsti = dst.astype(jnp.int32)
    d_pad, m_pad, partials = _sc_kernel(y, dsti, embeddings)
    total = _tc_kernel(d_pad.reshape(NROWS, 128), m_pad.reshape(NROWS, 128),
                       partials)
    return total[0, 0] / jnp.float32(N)
